# baseline (speedup 1.0000x reference)
import jax
import jax.numpy as jnp
from jax import lax
from jax.experimental import pallas as pl
from jax.experimental.pallas import tpu as pltpu
from jax.experimental.pallas import tpu_sc as plsc

B = 128
N = 32768
K = 16


_NC, _NS = 2, 16
_NW = _NC * _NS
_PER_W = (B * K) // _NW
_ROWS_W = B // _NW


def _sc_gather_body(te_hbm, sim_hbm, out_hbm, idx_v, val_v, sem):
    wid = lax.axis_index("s") * _NC + lax.axis_index("c")
    base = wid * _PER_W
    pltpu.sync_copy(te_hbm.at[pl.ds(base, _PER_W)], idx_v)
    for r in range(_ROWS_W):
        row = wid * _ROWS_W + r
        chunk = idx_v[pl.ds(r * K, 16)]
        idx_v[pl.ds(r * K, 16)] = chunk + row * N
    pltpu.async_copy(sim_hbm.at[idx_v], val_v, sem).wait()
    pltpu.sync_copy(val_v, out_hbm.at[pl.ds(base, _PER_W)])


def _sc_gather(te_flat, sim_flat):
    mesh = plsc.VectorSubcoreMesh(core_axis_name="c", subcore_axis_name="s")
    kfn = pl.kernel(
        _sc_gather_body,
        mesh=mesh,
        out_type=jax.ShapeDtypeStruct((B * K,), jnp.float32),
        scratch_types=[
            pltpu.VMEM((_PER_W,), jnp.int32),
            pltpu.VMEM((_PER_W,), jnp.float32),
            pltpu.SemaphoreType.DMA,
        ],
    )
    return kfn(te_flat, sim_flat)


_RB = 8
_STEPS = B // _RB
_SW = 32
_C126 = float(2.0 ** -126)
_C24 = float(2.0 ** 24)


def _stats_body(sim_ref, st_ref):
    x = sim_ref[...]
    y = 100.0 * x
    m = jnp.max(y, axis=1, keepdims=True)
    e = jnp.exp(y - m)
    z = jnp.sum(e, axis=1, keepdims=True)
    zz = z * _C126
    pos = (e * _C24) > zz
    nz = jnp.sum(jnp.where(pos, 1.0, 0.0), axis=1, keepdims=True)
    p16 = jnp.where(pos[:, :K], 1.0, 0.0)
    st = jnp.concatenate(
        [m, z, nz, jnp.zeros((_RB, 1), jnp.float32), p16,
         jnp.zeros((_RB, _SW - 4 - K), jnp.float32)], axis=1)
    st_ref[...] = st


def _tc_stats(similarity):
    return pl.pallas_call(
        _stats_body,
        grid=(_STEPS,),
        in_specs=[pl.BlockSpec((_RB, N), lambda b: (b, 0))],
        out_specs=pl.BlockSpec((_RB, _SW), lambda b: (b, 0)),
        out_shape=jax.ShapeDtypeStruct((B, _SW), jnp.float32),
    )(similarity)


def _decide_body(st_ref, xv_ref, te_ref, sim_ref, out_ref, inter_ref):
    b = pl.program_id(0)
    st = st_ref[...]
    m = st[:, 0:1]
    z = st[:, 1:2]
    nz = st[:, 2:3]
    p16 = st[:, 4:4 + K] > 0.5
    te = te_ref[...]
    yv = 100.0 * xv_ref[...]
    ev = jnp.exp(yv - m)
    vpos = (ev * _C24) > (z * _C126)

    colk = lax.broadcasted_iota(jnp.int32, (_RB, K), 1)
    dup = jnp.zeros((_RB, K), jnp.bool_)
    for j in range(K - 1):
        dup = dup | ((te == te[:, j:j + 1]) & (colk > j))

    nzb = jnp.zeros((_RB, K), jnp.float32)
    for j in range(K):
        nzb = nzb + jnp.where(p16[:, j:j + 1] & (j < te), 1.0, 0.0)
    tef = te.astype(jnp.float32)
    present0 = (te < K) & ((nz + tef - nzb) < float(K))
    inter_ref[...] = jnp.sum(
        jnp.where(present0 & (~dup), 1.0, 0.0), axis=1, keepdims=True)

    any_pos = jnp.any(vpos)

    @pl.when(any_pos)
    def _full():
        x = sim_ref[...]
        y = 100.0 * x
        e = jnp.exp(y - m)
        s = e / z
        sv = ev / z
        col = lax.broadcasted_iota(jnp.int32, (_RB, N), 1)
        inter = jnp.zeros((_RB, 1), jnp.float32)
        for i in range(K):
            svi = sv[:, i:i + 1]
            ti = te[:, i:i + 1]
            gt = jnp.sum(jnp.where(s > svi, 1.0, 0.0), axis=1, keepdims=True)
            eq = jnp.sum(
                jnp.where((s == svi) & (col < ti), 1.0, 0.0),
                axis=1, keepdims=True)
            present = (gt + eq) < float(K)
            inter = inter + jnp.where(present & (~dup[:, i:i + 1]), 1.0, 0.0)
        inter_ref[...] = inter

    acc = inter_ref[...] / float(K) * 100.0
    acc = jnp.round(acc * 1e6) / 1e6
    total = jnp.sum(acc, axis=0, keepdims=True)

    @pl.when(b == 0)
    def _init():
        out_ref[...] = jnp.zeros((1, 1), jnp.float32)

    out_ref[...] += total

    @pl.when(b == _STEPS - 1)
    def _fin():
        out_ref[...] = out_ref[...] / float(B)


def _tc_decide(stats, x_gathered, te, similarity):
    return pl.pallas_call(
        _decide_body,
        grid=(_STEPS,),
        in_specs=[
            pl.BlockSpec((_RB, _SW), lambda b: (b, 0)),
            pl.BlockSpec((_RB, K), lambda b: (b, 0)),
            pl.BlockSpec((_RB, K), lambda b: (b, 0)),
            pl.BlockSpec((_RB, N), lambda b: (b, 0)),
        ],
        out_specs=pl.BlockSpec((1, 1), lambda b: (0, 0)),
        out_shape=jax.ShapeDtypeStruct((1, 1), jnp.float32),
        scratch_shapes=[pltpu.VMEM((_RB, 1), jnp.float32)],
    )(stats, x_gathered, te, similarity)


def kernel(similarity, text_en, text_input):
    del text_input
    te = text_en.astype(jnp.int32)
    stats = _tc_stats(similarity)
    gathered = _sc_gather(te.reshape(-1), similarity.reshape(-1))
    out = _tc_decide(stats, gathered.reshape(B, K), te, similarity)
    return out.reshape(())


# speedup vs baseline: 1.1907x; 1.1907x over previous
import jax
import jax.numpy as jnp
from jax import lax
from jax.experimental import pallas as pl
from jax.experimental.pallas import tpu as pltpu
from jax.experimental.pallas import tpu_sc as plsc

B = 128
N = 32768
K = 16


_NC, _NS = 2, 16
_NW = _NC * _NS
_PER_W = (B * K) // _NW
_ROWS_W = B // _NW


def _sc_gather_body(te_hbm, sim_hbm, out_hbm, idx_v, val_v, sem):
    wid = lax.axis_index("s") * _NC + lax.axis_index("c")
    base = wid * _PER_W
    pltpu.sync_copy(te_hbm.at[pl.ds(base, _PER_W)], idx_v)
    for r in range(_ROWS_W):
        row = wid * _ROWS_W + r
        chunk = idx_v[pl.ds(r * K, 16)]
        idx_v[pl.ds(r * K, 16)] = chunk + row * N
    pltpu.async_copy(sim_hbm.at[idx_v], val_v, sem).wait()
    pltpu.sync_copy(val_v, out_hbm.at[pl.ds(base, _PER_W)])


def _sc_gather(te_flat, sim_flat):
    mesh = plsc.VectorSubcoreMesh(core_axis_name="c", subcore_axis_name="s")
    kfn = pl.kernel(
        _sc_gather_body,
        mesh=mesh,
        out_type=jax.ShapeDtypeStruct((B * K,), jnp.float32),
        scratch_types=[
            pltpu.VMEM((_PER_W,), jnp.int32),
            pltpu.VMEM((_PER_W,), jnp.float32),
            pltpu.SemaphoreType.DMA,
        ],
    )
    return kfn(te_flat, sim_flat)


_RB = 8
_STEPS = B // _RB
_SW = 32
_C126 = float(2.0 ** -126)
_C24 = float(2.0 ** 24)


def _stats_body(sim_ref, st_ref, flat_ref):
    x = sim_ref[...]
    y = 100.0 * x
    m = jnp.max(y, axis=1, keepdims=True)
    e = jnp.exp(y - m)
    z = jnp.sum(e, axis=1, keepdims=True)
    zz = z * _C126
    pos = (e * _C24) > zz
    nz = jnp.sum(jnp.where(pos, 1.0, 0.0), axis=1, keepdims=True)
    p16 = jnp.where(pos[:, :K], 1.0, 0.0)
    st_ref[...] = jnp.concatenate(
        [m, z, nz, jnp.zeros((_RB, 1), jnp.float32), p16,
         jnp.zeros((_RB, _SW - 4 - K), jnp.float32)], axis=1)
    flat_ref[...] = x.reshape(_RB * N)


def _tc_stats(similarity):
    return pl.pallas_call(
        _stats_body,
        grid=(_STEPS,),
        in_specs=[pl.BlockSpec((_RB, N), lambda b: (b, 0))],
        out_specs=[
            pl.BlockSpec((_RB, _SW), lambda b: (b, 0)),
            pl.BlockSpec((_RB * N,), lambda b: (b,)),
        ],
        out_shape=[
            jax.ShapeDtypeStruct((B, _SW), jnp.float32),
            jax.ShapeDtypeStruct((B * N,), jnp.float32),
        ],
    )(similarity)


def _decide_body(st_ref, xv_ref, te_ref, sim_hbm, out_ref, inter_ref,
                 xblk_ref, sem):
    b = pl.program_id(0)
    st = st_ref[...]
    m = st[:, 0:1]
    z = st[:, 1:2]
    nz = st[:, 2:3]
    p16 = st[:, 4:4 + K] > 0.5
    te = te_ref[...]
    yv = 100.0 * xv_ref[...]
    ev = jnp.exp(yv - m)
    vpos = (ev * _C24) > (z * _C126)

    colk = lax.broadcasted_iota(jnp.int32, (_RB, K), 1)
    dup = jnp.zeros((_RB, K), jnp.bool_)
    for j in range(K - 1):
        dup = dup | ((te == te[:, j:j + 1]) & (colk > j))

    nzb = jnp.zeros((_RB, K), jnp.float32)
    for j in range(K):
        nzb = nzb + jnp.where(p16[:, j:j + 1] & (j < te), 1.0, 0.0)
    tef = te.astype(jnp.float32)
    present0 = (te < K) & ((nz + tef - nzb) < float(K))
    inter_ref[...] = jnp.sum(
        jnp.where(present0 & (~dup), 1.0, 0.0), axis=1, keepdims=True)

    any_pos = jnp.any(vpos)

    @pl.when(any_pos)
    def _full():
        cp = pltpu.make_async_copy(
            sim_hbm.at[pl.ds(b * _RB, _RB), :], xblk_ref, sem)
        cp.start()
        cp.wait()
        x = xblk_ref[...]
        y = 100.0 * x
        e = jnp.exp(y - m)
        s = e / z
        sv = ev / z
        col = lax.broadcasted_iota(jnp.int32, (_RB, N), 1)
        inter = jnp.zeros((_RB, 1), jnp.float32)
        for i in range(K):
            svi = sv[:, i:i + 1]
            ti = te[:, i:i + 1]
            gt = jnp.sum(jnp.where(s > svi, 1.0, 0.0), axis=1, keepdims=True)
            eq = jnp.sum(
                jnp.where((s == svi) & (col < ti), 1.0, 0.0),
                axis=1, keepdims=True)
            present = (gt + eq) < float(K)
            inter = inter + jnp.where(present & (~dup[:, i:i + 1]), 1.0, 0.0)
        inter_ref[...] = inter

    acc = inter_ref[...] / float(K) * 100.0
    acc = jnp.round(acc * 1e6) / 1e6
    total = jnp.sum(acc, axis=0, keepdims=True)

    @pl.when(b == 0)
    def _init():
        out_ref[...] = jnp.zeros((1, 1), jnp.float32)

    out_ref[...] += total

    @pl.when(b == _STEPS - 1)
    def _fin():
        out_ref[...] = out_ref[...] / float(B)


def _tc_decide(stats, x_gathered, te, similarity):
    return pl.pallas_call(
        _decide_body,
        grid=(_STEPS,),
        in_specs=[
            pl.BlockSpec((_RB, _SW), lambda b: (b, 0)),
            pl.BlockSpec((_RB, K), lambda b: (b, 0)),
            pl.BlockSpec((_RB, K), lambda b: (b, 0)),
            pl.BlockSpec(memory_space=pl.ANY),
        ],
        out_specs=pl.BlockSpec((1, 1), lambda b: (0, 0)),
        out_shape=jax.ShapeDtypeStruct((1, 1), jnp.float32),
        scratch_shapes=[
            pltpu.VMEM((_RB, 1), jnp.float32),
            pltpu.VMEM((_RB, N), jnp.float32),
            pltpu.SemaphoreType.DMA,
        ],
    )(stats, x_gathered, te, similarity)


def kernel(similarity, text_en, text_input):
    del text_input
    te = text_en.astype(jnp.int32)
    stats, flat = _tc_stats(similarity)
    gathered = _sc_gather(te.reshape(-1), flat)
    out = _tc_decide(stats, gathered.reshape(B, K), te, similarity)
    return out.reshape(())


# speedup vs baseline: 1.5791x; 1.3261x over previous
import jax
import jax.numpy as jnp
from jax import lax
from jax.experimental import pallas as pl
from jax.experimental.pallas import tpu as pltpu
from jax.experimental.pallas import tpu_sc as plsc

B = 128
N = 32768
K = 16


_NC, _NS = 2, 16
_NW = _NC * _NS
_PER_W = (B * K) // _NW
_ROWS_W = B // _NW


def _sc_gather_body(te_hbm, sim_hbm, out_hbm, idx_v, val_v, sem):
    wid = lax.axis_index("s") * _NC + lax.axis_index("c")
    base = wid * _PER_W
    pltpu.sync_copy(te_hbm.at[pl.ds(base, _PER_W)], idx_v)
    for r in range(_ROWS_W):
        row = wid * _ROWS_W + r
        chunk = idx_v[pl.ds(r * K, 16)]
        idx_v[pl.ds(r * K, 16)] = chunk + row * N
    pltpu.async_copy(sim_hbm.at[idx_v], val_v, sem).wait()
    pltpu.sync_copy(val_v, out_hbm.at[pl.ds(base, _PER_W)])


def _sc_gather(te_flat, sim_flat):
    mesh = plsc.VectorSubcoreMesh(core_axis_name="c", subcore_axis_name="s")
    kfn = pl.kernel(
        _sc_gather_body,
        mesh=mesh,
        out_type=jax.ShapeDtypeStruct((B * K,), jnp.float32),
        scratch_types=[
            pltpu.VMEM((_PER_W,), jnp.int32),
            pltpu.VMEM((_PER_W,), jnp.float32),
            pltpu.SemaphoreType.DMA,
        ],
    )
    return kfn(te_flat, sim_flat)


_RA = 32
_ASTEPS = B // _RA
_RB = 8
_STEPS = B // _RB
_SW = 32
_C126 = float(2.0 ** -126)
_C24 = float(2.0 ** 24)


def _stats_body(sim_ref, st_ref, flat_ref):
    x = sim_ref[...]
    y = 100.0 * x
    m = jnp.max(y, axis=1, keepdims=True)
    e = jnp.exp(y - m)
    z = jnp.sum(e, axis=1, keepdims=True)
    zz = z * _C126
    pos = (e * _C24) > zz
    nz = jnp.sum(jnp.where(pos, 1.0, 0.0), axis=1, keepdims=True)
    p16 = jnp.where(pos[:, :K], 1.0, 0.0)
    st_ref[...] = jnp.concatenate(
        [m, z, nz, jnp.zeros((_RA, 1), jnp.float32), p16,
         jnp.zeros((_RA, _SW - 4 - K), jnp.float32)], axis=1)
    flat_ref[...] = x.reshape(_RA * N)


def _tc_stats(similarity):
    return pl.pallas_call(
        _stats_body,
        grid=(_ASTEPS,),
        in_specs=[pl.BlockSpec((_RA, N), lambda b: (b, 0))],
        out_specs=[
            pl.BlockSpec((_RA, _SW), lambda b: (b, 0)),
            pl.BlockSpec((_RA * N,), lambda b: (b,)),
        ],
        out_shape=[
            jax.ShapeDtypeStruct((B, _SW), jnp.float32),
            jax.ShapeDtypeStruct((B * N,), jnp.float32),
        ],
    )(similarity)


def _dup_mask(te):
    rows = te.shape[0]
    colk = lax.broadcasted_iota(jnp.int32, (rows, K), 1)
    dup = jnp.zeros((rows, K), jnp.bool_)
    for j in range(K - 1):
        dup = dup | ((te == te[:, j:j + 1]) & (colk > j))
    return dup


def _decide_body(st_ref, xv_ref, te_ref, sim_hbm, out_ref,
                 inter_ref, flags_ref, xblk_ref, sem):
    b = pl.program_id(0)

    @pl.when(b == 0)
    def _cheap_all():
        st = st_ref[...]
        m = st[:, 0:1]
        z = st[:, 1:2]
        nz = st[:, 2:3]
        p16 = st[:, 4:4 + K] > 0.5
        te = te_ref[...]
        yv = 100.0 * xv_ref[...]
        ev = jnp.exp(yv - m)
        vpos = (ev * _C24) > (z * _C126)
        dup = _dup_mask(te)
        nzb = jnp.zeros((B, K), jnp.float32)
        for j in range(K):
            nzb = nzb + jnp.where(p16[:, j:j + 1] & (j < te), 1.0, 0.0)
        tef = te.astype(jnp.float32)
        present0 = (te < K) & ((nz + tef - nzb) < float(K))
        inter_ref[...] = jnp.sum(
            jnp.where(present0 & (~dup), 1.0, 0.0), axis=1, keepdims=True)
        rowany = jnp.any(vpos, axis=1, keepdims=True)
        flags_ref[...] = jnp.max(
            jnp.where(rowany, 1.0, 0.0).reshape(_STEPS, _RB),
            axis=1, keepdims=True)

    flag = jnp.any(flags_ref[pl.ds(b, 1), :] > 0.0)

    @pl.when(flag)
    def _full():
        cp = pltpu.make_async_copy(
            sim_hbm.at[pl.ds(b * _RB, _RB), :], xblk_ref, sem)
        cp.start()
        cp.wait()
        st = st_ref[pl.ds(b * _RB, _RB), :]
        m = st[:, 0:1]
        z = st[:, 1:2]
        te = te_ref[pl.ds(b * _RB, _RB), :]
        xv = xv_ref[pl.ds(b * _RB, _RB), :]
        ev = jnp.exp(100.0 * xv - m)
        sv = ev / z
        dup = _dup_mask(te)
        x = xblk_ref[...]
        e = jnp.exp(100.0 * x - m)
        s = e / z
        col = lax.broadcasted_iota(jnp.int32, (_RB, N), 1)
        inter = jnp.zeros((_RB, 1), jnp.float32)
        for i in range(K):
            svi = sv[:, i:i + 1]
            ti = te[:, i:i + 1]
            gt = jnp.sum(jnp.where(s > svi, 1.0, 0.0), axis=1, keepdims=True)
            eq = jnp.sum(
                jnp.where((s == svi) & (col < ti), 1.0, 0.0),
                axis=1, keepdims=True)
            present = (gt + eq) < float(K)
            inter = inter + jnp.where(present & (~dup[:, i:i + 1]), 1.0, 0.0)
        inter_ref[pl.ds(b * _RB, _RB), :] = inter

    @pl.when(b == _STEPS - 1)
    def _fin():
        acc = inter_ref[...] / float(K) * 100.0
        acc = (acc * 1e6) / 1e6
        out_ref[...] = jnp.sum(acc, axis=0, keepdims=True) / float(B)


def _tc_decide(stats, x_gathered, te, similarity):
    return pl.pallas_call(
        _decide_body,
        grid=(_STEPS,),
        in_specs=[
            pl.BlockSpec((B, _SW), lambda b: (0, 0)),
            pl.BlockSpec((B, K), lambda b: (0, 0)),
            pl.BlockSpec((B, K), lambda b: (0, 0)),
            pl.BlockSpec(memory_space=pl.ANY),
        ],
        out_specs=pl.BlockSpec((1, 1), lambda b: (0, 0)),
        out_shape=jax.ShapeDtypeStruct((1, 1), jnp.float32),
        scratch_shapes=[
            pltpu.VMEM((B, 1), jnp.float32),
            pltpu.VMEM((_STEPS, 1), jnp.float32),
            pltpu.VMEM((_RB, N), jnp.float32),
            pltpu.SemaphoreType.DMA,
        ],
    )(stats, x_gathered, te, similarity)


def kernel(similarity, text_en, text_input):
    del text_input
    te = text_en.astype(jnp.int32)
    stats, flat = _tc_stats(similarity)
    gathered = _sc_gather(te.reshape(-1), flat)
    out = _tc_decide(stats, gathered.reshape(B, K), te, similarity)
    return out.reshape(())


# speedup vs baseline: 1.5845x; 1.0034x over previous
import jax
import jax.numpy as jnp
from jax import lax
from jax.experimental import pallas as pl
from jax.experimental.pallas import tpu as pltpu
from jax.experimental.pallas import tpu_sc as plsc

B = 128
N = 32768
K = 16

_W = N // K
_WMASK = _W - 1
_WSH = 11


_NC, _NS = 2, 16
_NW = _NC * _NS
_PER_W = (B * K) // _NW
_ROWS_W = B // _NW


def _sc_gather_body(te_hbm, words_hbm, out_hbm, idx_v, val_v, sem):
    wid = lax.axis_index("s") * _NC + lax.axis_index("c")
    base = wid * _PER_W
    pltpu.sync_copy(te_hbm.at[pl.ds(base, _PER_W)], idx_v)
    for r in range(_ROWS_W):
        row = wid * _ROWS_W + r
        chunk = idx_v[pl.ds(r * K, 16)]
        idx_v[pl.ds(r * K, 16)] = (chunk & _WMASK) + row * _W
    pltpu.async_copy(words_hbm.at[idx_v], val_v, sem).wait()
    pltpu.sync_copy(val_v, out_hbm.at[pl.ds(base, _PER_W)])


def _sc_gather(te_flat, words_flat):
    mesh = plsc.VectorSubcoreMesh(core_axis_name="c", subcore_axis_name="s")
    kfn = pl.kernel(
        _sc_gather_body,
        mesh=mesh,
        out_type=jax.ShapeDtypeStruct((B * K,), jnp.float32),
        scratch_types=[
            pltpu.VMEM((_PER_W,), jnp.int32),
            pltpu.VMEM((_PER_W,), jnp.float32),
            pltpu.SemaphoreType.DMA,
        ],
    )
    return kfn(te_flat, words_flat)


_RA = 32
_ASTEPS = B // _RA
_RB = 8
_STEPS = B // _RB
_SW = 32
_C126 = float(2.0 ** -126)
_C24 = float(2.0 ** 24)


def _stats_body(sim_ref, st_ref, words_ref):
    x = sim_ref[...]
    y = 100.0 * x
    m = jnp.max(y, axis=1, keepdims=True)
    e = jnp.exp(y - m)
    z = jnp.sum(e, axis=1, keepdims=True)
    zz = z * _C126
    pos = (e * _C24) > zz
    nz = jnp.sum(jnp.where(pos, 1.0, 0.0), axis=1, keepdims=True)
    p16 = jnp.where(pos[:, :K], 1.0, 0.0)
    st_ref[...] = jnp.concatenate(
        [m, z, nz, jnp.zeros((_RA, 1), jnp.float32), p16,
         jnp.zeros((_RA, _SW - 4 - K), jnp.float32)], axis=1)
    w = jnp.zeros((_RA, _W), jnp.float32)
    for j in range(K):
        w = w + jnp.where(pos[:, _W * j:_W * (j + 1)], float(2 ** j), 0.0)
    words_ref[...] = w.reshape(_RA * _W)


def _tc_stats(similarity):
    return pl.pallas_call(
        _stats_body,
        grid=(_ASTEPS,),
        in_specs=[pl.BlockSpec((_RA, N), lambda b: (b, 0))],
        out_specs=[
            pl.BlockSpec((_RA, _SW), lambda b: (b, 0)),
            pl.BlockSpec((_RA * _W,), lambda b: (b,)),
        ],
        out_shape=[
            jax.ShapeDtypeStruct((B, _SW), jnp.float32),
            jax.ShapeDtypeStruct((B * _W,), jnp.float32),
        ],
    )(similarity)


def _dup_mask(te):
    rows = te.shape[0]
    colk = lax.broadcasted_iota(jnp.int32, (rows, K), 1)
    dup = jnp.zeros((rows, K), jnp.bool_)
    for j in range(K - 1):
        dup = dup | ((te == te[:, j:j + 1]) & (colk > j))
    return dup


def _decide_body(st_ref, w_ref, te_ref, sim_hbm, out_ref,
                 inter_ref, flags_ref, xblk_ref, sem):
    b = pl.program_id(0)

    @pl.when(b == 0)
    def _cheap_all():
        st = st_ref[...]
        nz = st[:, 2:3]
        p16 = st[:, 4:4 + K] > 0.5
        te = te_ref[...]
        wv = w_ref[...].astype(jnp.int32)
        vpos = jnp.bitwise_and(
            jnp.right_shift(wv, jnp.right_shift(te, _WSH)), 1) > 0
        dup = _dup_mask(te)
        nzb = jnp.zeros((B, K), jnp.float32)
        for j in range(K):
            nzb = nzb + jnp.where(p16[:, j:j + 1] & (j < te), 1.0, 0.0)
        tef = te.astype(jnp.float32)
        present0 = (te < K) & ((nz + tef - nzb) < float(K))
        inter_ref[...] = jnp.sum(
            jnp.where(present0 & (~dup), 1.0, 0.0), axis=1, keepdims=True)
        rowany = jnp.any(vpos, axis=1, keepdims=True)
        flags_ref[...] = jnp.max(
            jnp.where(rowany, 1.0, 0.0).reshape(_STEPS, _RB),
            axis=1, keepdims=True)

    flag = jnp.any(flags_ref[pl.ds(b, 1), :] > 0.0)

    @pl.when(flag)
    def _full():
        cp = pltpu.make_async_copy(
            sim_hbm.at[pl.ds(b * _RB, _RB), :], xblk_ref, sem)
        cp.start()
        cp.wait()
        st = st_ref[pl.ds(b * _RB, _RB), :]
        m = st[:, 0:1]
        z = st[:, 1:2]
        te = te_ref[pl.ds(b * _RB, _RB), :]
        dup = _dup_mask(te)
        x = xblk_ref[...]
        e = jnp.exp(100.0 * x - m)
        s = e / z
        col = lax.broadcasted_iota(jnp.int32, (_RB, N), 1)
        inter = jnp.zeros((_RB, 1), jnp.float32)
        for i in range(K):
            ti = te[:, i:i + 1]
            hit = col == ti
            svi = jnp.sum(jnp.where(hit, s, 0.0), axis=1, keepdims=True)
            gt = jnp.sum(jnp.where(s > svi, 1.0, 0.0), axis=1, keepdims=True)
            eq = jnp.sum(
                jnp.where((s == svi) & (col < ti), 1.0, 0.0),
                axis=1, keepdims=True)
            present = (gt + eq) < float(K)
            inter = inter + jnp.where(present & (~dup[:, i:i + 1]), 1.0, 0.0)
        inter_ref[pl.ds(b * _RB, _RB), :] = inter

    @pl.when(b == _STEPS - 1)
    def _fin():
        acc = inter_ref[...] / float(K) * 100.0
        acc = (acc * 1e6) / 1e6
        out_ref[...] = jnp.sum(acc, axis=0, keepdims=True) / float(B)


def _tc_decide(stats, words_gathered, te, similarity):
    return pl.pallas_call(
        _decide_body,
        grid=(_STEPS,),
        in_specs=[
            pl.BlockSpec((B, _SW), lambda b: (0, 0)),
            pl.BlockSpec((B, K), lambda b: (0, 0)),
            pl.BlockSpec((B, K), lambda b: (0, 0)),
            pl.BlockSpec(memory_space=pl.ANY),
        ],
        out_specs=pl.BlockSpec((1, 1), lambda b: (0, 0)),
        out_shape=jax.ShapeDtypeStruct((1, 1), jnp.float32),
        scratch_shapes=[
            pltpu.VMEM((B, 1), jnp.float32),
            pltpu.VMEM((_STEPS, 1), jnp.float32),
            pltpu.VMEM((_RB, N), jnp.float32),
            pltpu.SemaphoreType.DMA,
        ],
    )(stats, words_gathered, te, similarity)


def kernel(similarity, text_en, text_input):
    del text_input
    te = text_en.astype(jnp.int32)
    stats, words = _tc_stats(similarity)
    gathered = _sc_gather(te.reshape(-1), words)
    out = _tc_decide(stats, gathered.reshape(B, K), te, similarity)
    return out.reshape(())


# speedup vs baseline: 1.5957x; 1.0071x over previous
import jax
import jax.numpy as jnp
from jax import lax
from jax.experimental import pallas as pl
from jax.experimental.pallas import tpu as pltpu
from jax.experimental.pallas import tpu_sc as plsc

B = 128
N = 32768
K = 16

_W = N // K
_WMASK = _W - 1
_WSH = 11


_NC, _NS = 2, 16
_NW = _NC * _NS
_PER_W = (B * K) // _NW
_ROWS_W = B // _NW


def _sc_gather_body(te_hbm, words_hbm, out_hbm, idx_v, val_v, sem):
    wid = lax.axis_index("s") * _NC + lax.axis_index("c")
    base = wid * _PER_W
    pltpu.sync_copy(te_hbm.at[pl.ds(base, _PER_W)], idx_v)
    for r in range(_ROWS_W):
        row = wid * _ROWS_W + r
        chunk = idx_v[pl.ds(r * K, 16)]
        idx_v[pl.ds(r * K, 16)] = (chunk & _WMASK) + row * _W
    pltpu.async_copy(words_hbm.at[idx_v], val_v, sem).wait()
    pltpu.sync_copy(val_v, out_hbm.at[pl.ds(base, _PER_W)])


def _sc_gather(te_flat, words_flat):
    mesh = plsc.VectorSubcoreMesh(core_axis_name="c", subcore_axis_name="s")
    kfn = pl.kernel(
        _sc_gather_body,
        mesh=mesh,
        out_type=jax.ShapeDtypeStruct((B * K,), jnp.float32),
        scratch_types=[
            pltpu.VMEM((_PER_W,), jnp.int32),
            pltpu.VMEM((_PER_W,), jnp.float32),
            pltpu.SemaphoreType.DMA,
        ],
    )
    return kfn(te_flat, words_flat)


_RA = 32
_ASTEPS = B // _RA
_RB = 8
_STEPS = B // _RB
_SW = 32
_C126 = float(2.0 ** -126)
_C24 = float(2.0 ** 24)


def _stats_body(sim_ref, st_ref, words_ref):
    x = sim_ref[...]
    y = 100.0 * x
    m = jnp.max(y, axis=1, keepdims=True)
    e = jnp.exp(y - m)
    z = jnp.sum(e, axis=1, keepdims=True)
    zz = z * _C126
    pos = (e * _C24) > zz
    nz = jnp.sum(jnp.where(pos, 1.0, 0.0), axis=1, keepdims=True)
    p16 = jnp.where(pos[:, :K], 1.0, 0.0)
    st_ref[...] = jnp.concatenate(
        [m, z, nz, jnp.zeros((_RA, 1), jnp.float32), p16,
         jnp.zeros((_RA, _SW - 4 - K), jnp.float32)], axis=1)
    w = jnp.zeros((_RA, _W), jnp.float32)
    for j in range(K):
        w = w + jnp.where(pos[:, _W * j:_W * (j + 1)], float(2 ** j), 0.0)
    words_ref[...] = w.reshape(_RA * _W)


def _tc_stats(similarity):
    return pl.pallas_call(
        _stats_body,
        grid=(_ASTEPS,),
        in_specs=[pl.BlockSpec((_RA, N), lambda b: (b, 0))],
        out_specs=[
            pl.BlockSpec((_RA, _SW), lambda b: (b, 0)),
            pl.BlockSpec((_RA * _W,), lambda b: (b,)),
        ],
        out_shape=[
            jax.ShapeDtypeStruct((B, _SW), jnp.float32),
            jax.ShapeDtypeStruct((B * _W,), jnp.float32),
        ],
    )(similarity)


def _dup_mask(te):
    rows = te.shape[0]
    colk = lax.broadcasted_iota(jnp.int32, (rows, K), 1)
    dup = jnp.zeros((rows, K), jnp.bool_)
    for j in range(K - 1):
        dup = dup | ((te == te[:, j:j + 1]) & (colk > j))
    return dup


def _decide_body(st_ref, w_ref, te_ref, sim_hbm, out_ref,
                 inter_ref, flags_ref, xblk_ref, sem):
    st = st_ref[...]
    nz = st[:, 2:3]
    p16 = st[:, 4:4 + K] > 0.5
    te = te_ref[...]
    wv = w_ref[...].astype(jnp.int32)
    vpos = jnp.bitwise_and(
        jnp.right_shift(wv, jnp.right_shift(te, _WSH)), 1) > 0
    dup = _dup_mask(te)
    nzb = jnp.zeros((B, K), jnp.float32)
    for j in range(K):
        nzb = nzb + jnp.where(p16[:, j:j + 1] & (j < te), 1.0, 0.0)
    tef = te.astype(jnp.float32)
    present0 = (te < K) & ((nz + tef - nzb) < float(K))
    inter_ref[...] = jnp.sum(
        jnp.where(present0 & (~dup), 1.0, 0.0), axis=1, keepdims=True)
    rowany = jnp.any(vpos, axis=1, keepdims=True)
    flags_ref[...] = jnp.max(
        jnp.where(rowany, 1.0, 0.0).reshape(_STEPS, _RB),
        axis=1, keepdims=True)

    def _block(b, carry):
        flag = jnp.any(flags_ref[pl.ds(b, 1), :] > 0.0)

        @pl.when(flag)
        def _full():
            cp = pltpu.make_async_copy(
                sim_hbm.at[pl.ds(b * _RB, _RB), :], xblk_ref, sem)
            cp.start()
            cp.wait()
            stb = st_ref[pl.ds(b * _RB, _RB), :]
            m = stb[:, 0:1]
            z = stb[:, 1:2]
            teb = te_ref[pl.ds(b * _RB, _RB), :]
            dupb = _dup_mask(teb)
            x = xblk_ref[...]
            e = jnp.exp(100.0 * x - m)
            s = e / z
            col = lax.broadcasted_iota(jnp.int32, (_RB, N), 1)
            inter = jnp.zeros((_RB, 1), jnp.float32)
            for i in range(K):
                ti = teb[:, i:i + 1]
                svi = jnp.sum(
                    jnp.where(col == ti, s, 0.0), axis=1, keepdims=True)
                gt = jnp.sum(
                    jnp.where(s > svi, 1.0, 0.0), axis=1, keepdims=True)
                eq = jnp.sum(
                    jnp.where((s == svi) & (col < ti), 1.0, 0.0),
                    axis=1, keepdims=True)
                present = (gt + eq) < float(K)
                inter = inter + jnp.where(
                    present & (~dupb[:, i:i + 1]), 1.0, 0.0)
            inter_ref[pl.ds(b * _RB, _RB), :] = inter

        return carry

    lax.fori_loop(0, _STEPS, _block, 0)

    acc = inter_ref[...] / float(K) * 100.0
    acc = (acc * 1e6) / 1e6
    out_ref[...] = jnp.sum(acc, axis=0, keepdims=True) / float(B)


def _tc_decide(stats, words_gathered, te, similarity):
    return pl.pallas_call(
        _decide_body,
        grid=(1,),
        in_specs=[
            pl.BlockSpec((B, _SW), lambda b: (0, 0)),
            pl.BlockSpec((B, K), lambda b: (0, 0)),
            pl.BlockSpec((B, K), lambda b: (0, 0)),
            pl.BlockSpec(memory_space=pl.ANY),
        ],
        out_specs=pl.BlockSpec((1, 1), lambda b: (0, 0)),
        out_shape=jax.ShapeDtypeStruct((1, 1), jnp.float32),
        scratch_shapes=[
            pltpu.VMEM((B, 1), jnp.float32),
            pltpu.VMEM((_STEPS, 1), jnp.float32),
            pltpu.VMEM((_RB, N), jnp.float32),
            pltpu.SemaphoreType.DMA,
        ],
    )(stats, words_gathered, te, similarity)


def kernel(similarity, text_en, text_input):
    del text_input
    te = text_en.astype(jnp.int32)
    stats, words = _tc_stats(similarity)
    gathered = _sc_gather(te.reshape(-1), words)
    out = _tc_decide(stats, gathered.reshape(B, K), te, similarity)
    return out.reshape(())
